# Initial kernel scaffold; baseline (speedup 1.0000x reference)
import jax
import jax.numpy as jnp
from jax.experimental import pallas as pl


def kernel(trg_token_ids_batch):
    raise NotImplementedError("write your pallas kernel here")


# speedup vs baseline: 1.8853x; 1.8853x over previous
import functools

import jax
import jax.numpy as jnp
from jax.experimental import pallas as pl

PAD = 0
VOCAB = 100000
BATCH = 1024
TILE = 2048


def _onehot_body(ids_ref, out_ref):
    j = pl.program_id(0)
    ids = ids_ref[:]
    cols = jax.lax.broadcasted_iota(jnp.int32, (BATCH, TILE), 1) + j * TILE
    hit = (cols == ids) & (ids != PAD)
    out_ref[:] = hit.astype(jnp.float32)


@jax.jit
def kernel(trg_token_ids_batch):
    grid = (pl.cdiv(VOCAB, TILE),)
    return pl.pallas_call(
        _onehot_body,
        grid=grid,
        in_specs=[pl.BlockSpec((BATCH, 1), lambda j: (0, 0))],
        out_specs=pl.BlockSpec((BATCH, TILE), lambda j: (0, j)),
        out_shape=jax.ShapeDtypeStruct((BATCH, VOCAB), jnp.float32),
    )(trg_token_ids_batch)
